# baseline (speedup 1.0000x reference)
import functools

import jax
import jax.numpy as jnp
from jax import lax
from jax.experimental import pallas as pl
from jax.experimental.pallas import tpu as pltpu
from jax.experimental.pallas import tpu_sc as plsc

B = 16384
E = 64
L = 16
NC = 2
NS = 16
NW = NC * NS
BPW = B // NW


def _sc_gather_reduce(uidx, midx, user_emb, ubias, movie_emb, mbias):
    mesh = plsc.VectorSubcoreMesh(core_axis_name="c", subcore_axis_name="s")

    @functools.partial(
        pl.kernel,
        out_type=[
            jax.ShapeDtypeStruct((NW, L), jnp.float32),
            jax.ShapeDtypeStruct((B,), jnp.float32),
        ],
        mesh=mesh,
        compiler_params=pltpu.CompilerParams(use_tc_tiling_on_sc=False),
        scratch_types=[
            pltpu.VMEM((BPW,), jnp.int32),
            pltpu.VMEM((BPW,), jnp.int32),
            pltpu.VMEM((BPW, E), jnp.float32),
            pltpu.VMEM((BPW, E), jnp.float32),
            pltpu.VMEM((BPW,), jnp.float32),
            pltpu.VMEM((BPW,), jnp.float32),
            pltpu.VMEM((BPW,), jnp.float32),
            pltpu.VMEM((L,), jnp.float32),
            pltpu.SemaphoreType.DMA,
            pltpu.SemaphoreType.DMA,
            pltpu.SemaphoreType.DMA,
            pltpu.SemaphoreType.DMA,
        ],
    )
    def k(uidx_hbm, midx_hbm, uemb_hbm, ubias_hbm, memb_hbm, mbias_hbm,
          partials_hbm, bsum_hbm,
          uidx_v, midx_v, urows_v, mrows_v, ub_v, mb_v, bs_v, acc_v,
          sem0, sem1, sem2, sem3):
        wid = lax.axis_index("s") * NC + lax.axis_index("c")
        base = wid * BPW
        pltpu.sync_copy(uidx_hbm.at[pl.ds(base, BPW)], uidx_v)
        pltpu.sync_copy(midx_hbm.at[pl.ds(base, BPW)], midx_v)
        cp0 = pltpu.async_copy(uemb_hbm.at[uidx_v], urows_v, sem0)
        cp1 = pltpu.async_copy(memb_hbm.at[midx_v], mrows_v, sem1)
        cp2 = pltpu.async_copy(ubias_hbm.at[uidx_v], ub_v, sem2)
        cp3 = pltpu.async_copy(mbias_hbm.at[midx_v], mb_v, sem3)
        cp2.wait()
        cp3.wait()
        for c in range(BPW // L):
            bs_v[pl.ds(c * L, L)] = ub_v[pl.ds(c * L, L)] + mb_v[pl.ds(c * L, L)]
        pltpu.sync_copy(bs_v, bsum_hbm.at[pl.ds(base, BPW)])
        cp0.wait()
        cp1.wait()

        def body(i, acc):
            for j in range(E // L):
                acc = acc + urows_v[i, pl.ds(j * L, L)] * mrows_v[i, pl.ds(j * L, L)]
            return acc

        acc = lax.fori_loop(0, BPW, body, jnp.zeros((L,), jnp.float32))
        acc_v[...] = acc
        pltpu.sync_copy(acc_v, partials_hbm.at[wid])

    return k(uidx, midx, user_emb, ubias, movie_emb, mbias)


def _tc_finish(partials, bias2d):
    def body(p_ref, b_ref, o_ref):
        s = jnp.sum(p_ref[...])
        o_ref[...] = jax.nn.sigmoid(b_ref[...] + s)

    return pl.pallas_call(
        body,
        out_shape=jax.ShapeDtypeStruct(bias2d.shape, jnp.float32),
    )(partials, bias2d)


def kernel(inputs, user_emb, user_bias_tab, movie_emb, movie_bias_tab):
    uidx = inputs[:, 0]
    midx = inputs[:, 1]
    ubias = user_bias_tab[:, 0]
    mbias = movie_bias_tab[:, 0]
    partials, bsum = _sc_gather_reduce(uidx, midx, user_emb, ubias,
                                       movie_emb, mbias)
    out2d = _tc_finish(partials, bsum.reshape(128, 128))
    return out2d.reshape(B, 1)
